# baseline (speedup 1.0000x reference)
import numpy as np
import jax
import jax.numpy as jnp
from jax.experimental import pallas as pl
from jax.experimental.pallas import tpu as pltpu


def _gram_constants():
    Tt = np.zeros((64, 256), dtype=np.float32)
    Tr = np.zeros((64, 256), dtype=np.float32)
    S = np.zeros((256, 16), dtype=np.float32)
    for m in range(4):
        for n in range(4):
            for k in range(16):
                j = m * 64 + n * 16 + k
                Tt[n * 16 + k, j] = 1.0
                Tr[m * 16 + k, j] = 1.0
                S[j, n * 4 + m] = 1.0
    I2 = np.eye(2, dtype=np.float32)
    return np.kron(I2, Tt), np.kron(I2, Tr), np.kron(I2, S)


_TT_NP, _TR_NP, _S_NP = _gram_constants()


def _fused_kernel(x_ref, eps_ref,
                  Wa_ref, ba_ref, Wb_ref, bb_ref, Wc_ref, bc_ref,
                  Wmu_ref, bmu_ref, Wls_ref, bls_ref,
                  We_ref, be_ref, Wf1_ref, bf1_ref, Wf2_ref, bf2_ref,
                  Tt_ref, Tr_ref, S_ref,
                  xspv_ref, xspr_ref, adj_ref):
    f32 = jnp.float32
    dot = lambda a, b: jnp.dot(a, b, preferred_element_type=f32)
    x = x_ref[:]
    h1 = jnp.maximum(dot(x, Wa_ref[:]) + ba_ref[:], 0.0)
    nodef = dot(h1, Wb_ref[:]) + bb_ref[:]
    h = dot(nodef, Wc_ref[:]) + bc_ref[:]
    mu = dot(h, Wmu_ref[:]) + bmu_ref[:]
    logstd = dot(h, Wls_ref[:]) + bls_ref[:]
    z = mu + eps_ref[:] * jnp.exp(logstd)

    P = dot(z, Tt_ref[:]) * dot(z, Tr_ref[:])
    adj_ref[:] = jax.nn.sigmoid(dot(P, S_ref[:]))

    dh = jnp.maximum(dot(z, We_ref[:]) + be_ref[:], 0.0)
    xspv_ref[:] = dot(dh, Wf1_ref[:]) + bf1_ref[:]
    xspr_ref[:] = dot(dh, Wf2_ref[:]) + bf2_ref[:]


def kernel(dev_sh, dev_spv, dev_spr, dev_level, sh_W1, sh_b1, sh_W2, sh_b2,
           c1_W, c1_b, cmu_W, cmu_b, cls_W, cls_b, spv_W1, spv_b1, spv_W2,
           spv_b2, spr_W1, spr_b1, spr_W2, spr_b2, eps):
    B = dev_sh.shape[0]
    H = B // 2
    blk = 4096 if H % 4096 == 0 else H
    f32 = jnp.float32
    I4 = jnp.eye(4, dtype=f32)
    I2 = jnp.eye(2, dtype=f32)

    def bd(W):
        return jnp.kron(I4, W)

    def p2(W):
        return jnp.kron(I2, W)

    def tb(b):
        return jnp.tile(b, 8)[None, :]

    Wa, ba = p2(bd(sh_W1)), tb(sh_b1)
    Wb, bb = p2(bd(sh_W2)), tb(sh_b2)
    Wc, bc = p2(bd(c1_W)), tb(c1_b)
    Wmu, bmu = p2(bd(cmu_W)), tb(cmu_b)
    Wls, bls = p2(bd(cls_W)), tb(cls_b)
    z32 = jnp.zeros((64, 32), f32)
    We = p2(jnp.concatenate([bd(spv_W1), z32, bd(spr_W1), z32], axis=1))
    be1 = jnp.concatenate(
        [jnp.tile(spv_b1, 4), jnp.zeros((32,), f32), jnp.tile(spr_b1, 4),
         jnp.zeros((32,), f32)])
    be = jnp.tile(be1, 2)[None, :]
    zpad = jnp.zeros((32, 64), f32)
    Wf1 = p2(jnp.concatenate(
        [bd(spv_W2), zpad, jnp.zeros((96, 64), f32), zpad], axis=0))
    Wf2 = p2(jnp.concatenate(
        [jnp.zeros((96, 64), f32), zpad, bd(spr_W2), zpad], axis=0))
    bf1, bf2 = tb(spv_b2), tb(spr_b2)
    Tt, Tr, S = jnp.asarray(_TT_NP), jnp.asarray(_TR_NP), jnp.asarray(_S_NP)

    x12 = jnp.concatenate([dev_spv, dev_sh, dev_level, dev_spr], axis=1)
    x2 = x12.reshape(H, 24)
    eps2 = eps.reshape(H, 128)

    row_spec = lambda w: pl.BlockSpec((blk, w), lambda i: (i, 0))
    full = lambda a: pl.BlockSpec(a.shape, lambda i: (0,) * a.ndim)

    xspv, xspr, adj = pl.pallas_call(
        _fused_kernel,
        grid=(H // blk,),
        in_specs=[row_spec(24), row_spec(128),
                  full(Wa), full(ba), full(Wb), full(bb), full(Wc), full(bc),
                  full(Wmu), full(bmu), full(Wls), full(bls),
                  full(We), full(be), full(Wf1), full(bf1), full(Wf2),
                  full(bf2), full(Tt), full(Tr), full(S)],
        out_specs=[row_spec(128), row_spec(128), row_spec(32)],
        out_shape=[jax.ShapeDtypeStruct((H, 128), f32),
                   jax.ShapeDtypeStruct((H, 128), f32),
                   jax.ShapeDtypeStruct((H, 32), f32)],
        compiler_params=pltpu.CompilerParams(
            dimension_semantics=("parallel",)),
    )(x2, eps2,
      Wa, ba, Wb, bb, Wc, bc, Wmu, bmu, Wls, bls,
      We, be, Wf1, bf1, Wf2, bf2, Tt, Tr, S)

    return (xspv.reshape(B, 4, 16), xspr.reshape(B, 4, 16),
            adj.reshape(B, 4, 4))


# speedup vs baseline: 3.2860x; 3.2860x over previous
import numpy as np
import jax
import jax.numpy as jnp
from jax.experimental import pallas as pl
from jax.experimental.pallas import tpu as pltpu

_PAIRS = [(0, 1), (0, 2), (0, 3), (1, 2), (1, 3), (2, 3)]


def _gram_constants():
    Sd = np.zeros((64, 16), dtype=np.float32)
    for n in range(4):
        for k in range(16):
            Sd[n * 16 + k, n * 4 + n] = 1.0
    To1 = np.zeros((64, 96), dtype=np.float32)
    To2 = np.zeros((64, 96), dtype=np.float32)
    So = np.zeros((96, 16), dtype=np.float32)
    for p, (n, m) in enumerate(_PAIRS):
        for k in range(16):
            To1[n * 16 + k, p * 16 + k] = 1.0
            To2[m * 16 + k, p * 16 + k] = 1.0
            So[p * 16 + k, n * 4 + m] = 1.0
            So[p * 16 + k, m * 4 + n] = 1.0
    return Sd, To1, To2, So


_SD_NP, _TO1_NP, _TO2_NP, _SO_NP = _gram_constants()


def _fused_kernel(x_ref, eps_ref,
                  Wa_ref, ba_ref, Wb_ref, bb_ref, Wc_ref, bc_ref,
                  Wd_ref, bd_ref, We_ref, be_ref, Wf_ref, bf_ref,
                  Sd_ref, To1_ref, To2_ref, So_ref,
                  xspv_ref, xspr_ref, adj_ref):
    f32 = jnp.float32
    dot = lambda a, b: jnp.dot(a, b, preferred_element_type=f32)
    x = x_ref[:]
    h1 = jnp.maximum(dot(x, Wa_ref[:]) + ba_ref[:], 0.0)
    nodef = dot(h1, Wb_ref[:]) + bb_ref[:]
    h = dot(nodef, Wc_ref[:]) + bc_ref[:]
    ml = dot(h, Wd_ref[:]) + bd_ref[:]
    z = ml[:, :64] + eps_ref[:] * jnp.exp(ml[:, 64:])

    diag = dot(z * z, Sd_ref[:])
    P = dot(z, To1_ref[:]) * dot(z, To2_ref[:])
    adj_ref[:] = jax.nn.sigmoid(diag + dot(P, So_ref[:]))

    dh = jnp.maximum(dot(z, We_ref[:]) + be_ref[:], 0.0)
    out = dot(dh, Wf_ref[:]) + bf_ref[:]
    xspv_ref[:] = out[:, :64]
    xspr_ref[:] = out[:, 64:]


def kernel(dev_sh, dev_spv, dev_spr, dev_level, sh_W1, sh_b1, sh_W2, sh_b2,
           c1_W, c1_b, cmu_W, cmu_b, cls_W, cls_b, spv_W1, spv_b1, spv_W2,
           spv_b2, spr_W1, spr_b1, spr_W2, spr_b2, eps):
    B = dev_sh.shape[0]
    blk = 8192 if B % 8192 == 0 else B
    f32 = jnp.float32
    I4 = jnp.eye(4, dtype=f32)

    def bd(W):
        return jnp.kron(I4, W)

    def tb(b):
        return jnp.tile(b, 4)[None, :]

    Wa, ba = bd(sh_W1), tb(sh_b1)
    Wb, bb = bd(sh_W2), tb(sh_b2)
    Wc, bc = bd(c1_W), tb(c1_b)
    Wd = jnp.concatenate([bd(cmu_W), bd(cls_W)], axis=1)
    bdd = jnp.concatenate([tb(cmu_b), tb(cls_b)], axis=1)
    We = jnp.concatenate([bd(spv_W1), bd(spr_W1)], axis=1)
    be = jnp.concatenate([tb(spv_b1), tb(spr_b1)], axis=1)
    Wf = jnp.concatenate([
        jnp.concatenate([bd(spv_W2), jnp.zeros((96, 64), f32)], axis=1),
        jnp.concatenate([jnp.zeros((96, 64), f32), bd(spr_W2)], axis=1)],
        axis=0)
    bf = jnp.concatenate([tb(spv_b2), tb(spr_b2)], axis=1)
    Sd, To1 = jnp.asarray(_SD_NP), jnp.asarray(_TO1_NP)
    To2, So = jnp.asarray(_TO2_NP), jnp.asarray(_SO_NP)

    x12 = jnp.concatenate([dev_spv, dev_sh, dev_level, dev_spr], axis=1)
    eps2 = eps.reshape(B, 64)

    row_spec = lambda w: pl.BlockSpec((blk, w), lambda i: (i, 0))
    full = lambda a: pl.BlockSpec(a.shape, lambda i: (0,) * a.ndim)

    xspv, xspr, adj = pl.pallas_call(
        _fused_kernel,
        grid=(B // blk,),
        in_specs=[row_spec(12), row_spec(64),
                  full(Wa), full(ba), full(Wb), full(bb), full(Wc), full(bc),
                  full(Wd), full(bdd), full(We), full(be), full(Wf), full(bf),
                  full(Sd), full(To1), full(To2), full(So)],
        out_specs=[row_spec(64), row_spec(64), row_spec(16)],
        out_shape=[jax.ShapeDtypeStruct((B, 64), f32),
                   jax.ShapeDtypeStruct((B, 64), f32),
                   jax.ShapeDtypeStruct((B, 16), f32)],
        compiler_params=pltpu.CompilerParams(
            dimension_semantics=("parallel",)),
    )(x12, eps2,
      Wa, ba, Wb, bb, Wc, bc, Wd, bdd, We, be, Wf, bf, Sd, To1, To2, So)

    return (xspv.reshape(B, 4, 16), xspr.reshape(B, 4, 16),
            adj.reshape(B, 4, 4))


# speedup vs baseline: 3.3260x; 1.0122x over previous
import numpy as np
import jax
import jax.numpy as jnp
from jax.experimental import pallas as pl
from jax.experimental.pallas import tpu as pltpu

_PAIRS = [(0, 1), (0, 2), (0, 3), (1, 2), (1, 3), (2, 3)]


def _gram_constants():
    Sd = np.zeros((64, 16), dtype=np.float32)
    for n in range(4):
        for k in range(16):
            Sd[n * 16 + k, n * 4 + n] = 1.0
    To1 = np.zeros((64, 96), dtype=np.float32)
    To2 = np.zeros((64, 96), dtype=np.float32)
    So = np.zeros((96, 16), dtype=np.float32)
    for p, (n, m) in enumerate(_PAIRS):
        for k in range(16):
            To1[n * 16 + k, p * 16 + k] = 1.0
            To2[m * 16 + k, p * 16 + k] = 1.0
            So[p * 16 + k, n * 4 + m] = 1.0
            So[p * 16 + k, m * 4 + n] = 1.0
    return Sd, To1, To2, So


_SD_NP, _TO1_NP, _TO2_NP, _SO_NP = _gram_constants()


def _fused_kernel(x_ref, eps_ref,
                  Wa_ref, ba_ref, Wb_ref, bb_ref, Wc_ref, bc_ref,
                  Wd_ref, bd_ref, We_ref, be_ref, Wf_ref, bf_ref,
                  Sd_ref, To1_ref, To2_ref, So_ref,
                  xspv_ref, xspr_ref, adj_ref):
    f32 = jnp.float32
    bf = jnp.bfloat16
    dot = lambda a, b: jnp.dot(a.astype(bf), b, preferred_element_type=f32)
    x = x_ref[:]
    h1 = jnp.maximum(dot(x, Wa_ref[:]) + ba_ref[:], 0.0)
    nodef = dot(h1, Wb_ref[:]) + bb_ref[:]
    h = dot(nodef, Wc_ref[:]) + bc_ref[:]
    ml = dot(h, Wd_ref[:]) + bd_ref[:]
    z = ml[:, :64] + eps_ref[:] * jnp.exp(ml[:, 64:])

    diag = dot(z * z, Sd_ref[:])
    P = dot(z, To1_ref[:]) * dot(z, To2_ref[:])
    adj_ref[:] = jax.nn.sigmoid(diag + dot(P, So_ref[:]))

    dh = jnp.maximum(dot(z, We_ref[:]) + be_ref[:], 0.0)
    out = dot(dh, Wf_ref[:]) + bf_ref[:]
    xspv_ref[:] = out[:, :64]
    xspr_ref[:] = out[:, 64:]


def kernel(dev_sh, dev_spv, dev_spr, dev_level, sh_W1, sh_b1, sh_W2, sh_b2,
           c1_W, c1_b, cmu_W, cmu_b, cls_W, cls_b, spv_W1, spv_b1, spv_W2,
           spv_b2, spr_W1, spr_b1, spr_W2, spr_b2, eps):
    B = dev_sh.shape[0]
    blk = 8192 if B % 8192 == 0 else B
    f32 = jnp.float32
    bf16 = jnp.bfloat16
    I4 = jnp.eye(4, dtype=f32)

    def bd(W):
        return jnp.kron(I4, W)

    def tb(b):
        return jnp.tile(b, 4)[None, :]

    Wa, ba = bd(sh_W1), tb(sh_b1)
    Wb, bb = bd(sh_W2), tb(sh_b2)
    Wc, bc = bd(c1_W), tb(c1_b)
    Wd = jnp.concatenate([bd(cmu_W), bd(cls_W)], axis=1)
    bdd = jnp.concatenate([tb(cmu_b), tb(cls_b)], axis=1)
    We = jnp.concatenate([bd(spv_W1), bd(spr_W1)], axis=1)
    be = jnp.concatenate([tb(spv_b1), tb(spr_b1)], axis=1)
    Wf = jnp.concatenate([
        jnp.concatenate([bd(spv_W2), jnp.zeros((96, 64), f32)], axis=1),
        jnp.concatenate([jnp.zeros((96, 64), f32), bd(spr_W2)], axis=1)],
        axis=0)
    bf = jnp.concatenate([tb(spv_b2), tb(spr_b2)], axis=1)
    Wa, Wb, Wc, Wd, We, Wf = (w.astype(bf16) for w in (Wa, Wb, Wc, Wd, We, Wf))
    Sd, To1 = jnp.asarray(_SD_NP, bf16), jnp.asarray(_TO1_NP, bf16)
    To2, So = jnp.asarray(_TO2_NP, bf16), jnp.asarray(_SO_NP, bf16)

    x12 = jnp.concatenate(
        [dev_spv, dev_sh, dev_level, dev_spr], axis=1).astype(bf16)
    eps2 = eps.reshape(B, 64)

    row_spec = lambda w: pl.BlockSpec((blk, w), lambda i: (i, 0))
    full = lambda a: pl.BlockSpec(a.shape, lambda i: (0,) * a.ndim)

    xspv, xspr, adj = pl.pallas_call(
        _fused_kernel,
        grid=(B // blk,),
        in_specs=[row_spec(12), row_spec(64),
                  full(Wa), full(ba), full(Wb), full(bb), full(Wc), full(bc),
                  full(Wd), full(bdd), full(We), full(be), full(Wf), full(bf),
                  full(Sd), full(To1), full(To2), full(So)],
        out_specs=[row_spec(64), row_spec(64), row_spec(16)],
        out_shape=[jax.ShapeDtypeStruct((B, 64), f32),
                   jax.ShapeDtypeStruct((B, 64), f32),
                   jax.ShapeDtypeStruct((B, 16), f32)],
        compiler_params=pltpu.CompilerParams(
            dimension_semantics=("parallel",)),
    )(x12, eps2,
      Wa, ba, Wb, bb, Wc, bc, Wd, bdd, We, be, Wf, bf, Sd, To1, To2, So)

    return (xspv.reshape(B, 4, 16), xspr.reshape(B, 4, 16),
            adj.reshape(B, 4, 4))
